# Initial kernel scaffold; baseline (speedup 1.0000x reference)
import jax
import jax.numpy as jnp
from jax.experimental import pallas as pl


def kernel(rbf, new_node, W1, b1, W2, b2, src, dst, cluster_id):
    raise NotImplementedError("write your pallas kernel here")


# speedup vs baseline: 9.8636x; 9.8636x over previous
import functools

import jax
import jax.numpy as jnp
from jax import lax
from jax.experimental import pallas as pl
from jax.experimental.pallas import tpu as pltpu
from jax.experimental.pallas import tpu_sc as plsc

N = 50000
E = 800000
RBF = 16
D = 64
HALF = 25000
ACC_ROWS = 25088
DUMP = HALF
NSUB = 16
SPAN = E // NSUB
C2 = 2000
GW = 80
GJ = 16
CM = GJ * GW
NCH = E // CM
KMAX = -(-NCH // NSUB)
TILE_OUT = ACC_ROWS // NSUB

_MESH = dict(core_axis_name="c", subcore_axis_name="s", num_cores=2,
             num_subcores=NSUB)


def _mlp_body(rbf_ref, w1_ref, b1_ref, w2_ref, b2_ref, h_ref):
    x = jnp.dot(rbf_ref[...], w1_ref[...],
                preferred_element_type=jnp.float32) + b1_ref[...]
    bx = 0.5 * x
    safe = jnp.minimum(bx, 14.0)
    y = jnp.where(bx > 14.0, x, 2.0 * jnp.log1p(jnp.exp(safe)))
    h_ref[...] = jnp.dot(y, w2_ref[...],
                         preferred_element_type=jnp.float32) + b2_ref[...]


def _mlp(rbf, w1, b1, w2, b2):
    be = 2000
    rbf2 = rbf.reshape(E // 2, 2 * RBF)
    z1 = jnp.zeros_like(w1)
    w1b = jnp.concatenate(
        [jnp.concatenate([w1, z1], 1), jnp.concatenate([z1, w1], 1)], 0)
    z2 = jnp.zeros_like(w2)
    w2b = jnp.concatenate(
        [jnp.concatenate([w2, z2], 1), jnp.concatenate([z2, w2], 1)], 0)
    b1b = jnp.concatenate([b1, b1]).reshape(1, 2 * D)
    b2b = jnp.concatenate([b2, b2]).reshape(1, 2 * D)
    return pl.pallas_call(
        _mlp_body,
        grid=(E // 2 // be,),
        in_specs=[
            pl.BlockSpec((be, 2 * RBF), lambda i: (i, 0)),
            pl.BlockSpec((2 * RBF, 2 * D), lambda i: (0, 0)),
            pl.BlockSpec((1, 2 * D), lambda i: (0, 0)),
            pl.BlockSpec((2 * D, 2 * D), lambda i: (0, 0)),
            pl.BlockSpec((1, 2 * D), lambda i: (0, 0)),
        ],
        out_specs=pl.BlockSpec((be, 2 * D), lambda i: (i, 0)),
        out_shape=jax.ShapeDtypeStruct((E // 2, 2 * D), jnp.float32),
    )(rbf2, w1b, b1b, w2b, b2b)


@functools.partial(
    pl.kernel,
    out_type=(jax.ShapeDtypeStruct((E,), jnp.int32),
              jax.ShapeDtypeStruct((E,), jnp.int32)),
    mesh=plsc.VectorSubcoreMesh(**_MESH),
    scratch_types=[
        pltpu.VMEM((N,), jnp.int32),
        pltpu.VMEM((C2,), jnp.int32),
        pltpu.VMEM((C2,), jnp.int32),
        pltpu.VMEM((C2,), jnp.int32),
        pltpu.VMEM((C2,), jnp.int32),
        pltpu.VMEM((16,), jnp.int32),
        pltpu.VMEM((NSUB * 16,), jnp.int32),
        pltpu.VMEM_SHARED((NSUB * 16,), jnp.int32),
    ],
    compiler_params=pltpu.CompilerParams(needs_layout_passes=False),
)
def _mask_kernel(src_hbm, dst_hbm, cluster_hbm, idx0_hbm, idx1_hbm,
                 cluster_v, srcb, dstb, o0, o1, cnt_stage, cnt_v, counts_sp):
    c = lax.axis_index("c")
    s = lax.axis_index("s")
    e0 = s * SPAN

    @pl.when(c == 0)
    def _phase1():
        pltpu.sync_copy(cluster_hbm, cluster_v)

        def chunk_count(k, tot):
            off = e0 + k * C2
            pltpu.sync_copy(src_hbm.at[pl.ds(off, C2)], srcb)
            pltpu.sync_copy(dst_hbm.at[pl.ds(off, C2)], dstb)

            def grp(j, t):
                sv = srcb[pl.ds(j * 16, 16)]
                dv = dstb[pl.ds(j * 16, 16)]
                cs = plsc.load_gather(cluster_v, [sv])
                cd = plsc.load_gather(cluster_v, [dv])
                return t + jnp.where(cs != cd, 1, 0).astype(jnp.int32)

            return lax.fori_loop(0, C2 // 16, grp, tot)

        tot = lax.fori_loop(0, SPAN // C2, chunk_count,
                            jnp.zeros((16,), jnp.int32))
        cnt_stage[...] = jnp.broadcast_to(jnp.sum(tot), (16,))
        pltpu.sync_copy(cnt_stage, counts_sp.at[pl.ds(s * 16, 16)])

    plsc.subcore_barrier()

    @pl.when(c == 0)
    def _phase2():
        pltpu.sync_copy(counts_sp, cnt_v)

        def accum(t, carry):
            bvec, tvec = carry
            row = cnt_v[pl.ds(t * 16, 16)]
            return (bvec + jnp.where(t < s, row, 0), tvec + row)

        zero16 = jnp.zeros((16,), jnp.int32)
        bvec, tvec = lax.fori_loop(0, NSUB, accum, (zero16, zero16))
        base0 = bvec[0]
        total = tvec[0]
        nf = total.astype(jnp.float32) * jnp.float32(0.2)
        k0 = nf.astype(jnp.int32)
        n_sel = k0 - jnp.where(k0.astype(jnp.float32) > nf, 1, 0)

        def chunk2(k, base):
            off = e0 + k * C2
            pltpu.sync_copy(src_hbm.at[pl.ds(off, C2)], srcb)
            pltpu.sync_copy(dst_hbm.at[pl.ds(off, C2)], dstb)

            def grp(j, b):
                sv = srcb[pl.ds(j * 16, 16)]
                dv = dstb[pl.ds(j * 16, 16)]
                cs = plsc.load_gather(cluster_v, [sv])
                cd = plsc.load_gather(cluster_v, [dv])
                diff = cs != cd
                di = jnp.where(diff, 1, 0).astype(jnp.int32)
                csum = plsc.cumsum(di)
                keep = jnp.where(diff, (csum + b) <= n_sel, True)
                d_lt = dv < HALF
                o0[pl.ds(j * 16, 16)] = jnp.where(keep & d_lt, dv, DUMP)
                o1[pl.ds(j * 16, 16)] = jnp.where(keep & (~d_lt),
                                                  dv - HALF, DUMP)
                return b + csum[15]

            base = lax.fori_loop(0, C2 // 16, grp, base)
            pltpu.sync_copy(o0, idx0_hbm.at[pl.ds(off, C2)])
            pltpu.sync_copy(o1, idx1_hbm.at[pl.ds(off, C2)])
            return base

        lax.fori_loop(0, SPAN // C2, chunk2, base0)


@functools.partial(
    pl.kernel,
    out_type=jax.ShapeDtypeStruct((N, D), jnp.float32),
    mesh=plsc.VectorSubcoreMesh(**_MESH),
    scratch_types=[
        pltpu.VMEM((GW, D), jnp.float32),
        pltpu.VMEM((GW, D), jnp.float32),
        pltpu.VMEM((GW // 2, 2 * D), jnp.float32),
        pltpu.VMEM((GW // 2, 2 * D), jnp.float32),
        pltpu.VMEM((16, D), jnp.float32),
        pltpu.VMEM((CM,), jnp.int32),
        pltpu.VMEM((CM,), jnp.int32),
        pltpu.VMEM_SHARED((ACC_ROWS, D), jnp.float32),
        pltpu.SemaphoreType.DMA,
        pltpu.SemaphoreType.DMA,
    ],
    compiler_params=pltpu.CompilerParams(needs_layout_passes=False,
                                         use_tc_tiling_on_sc=False),
)
def _main_kernel(h_hbm, nn_hbm, src2_hbm, idx02_hbm, idx12_hbm, outp_hbm,
                 rows0, rows1, h0, h1, zero_v, srcv, idxv, acc_sp, gsem, ssem):
    c = lax.axis_index("c")
    s = lax.axis_index("s")

    def zrow(i, _):
        for jj in range(D // 16):
            zero_v[i, pl.ds(jj * 16, 16)] = jnp.zeros((16,), jnp.float32)
        return 0

    lax.fori_loop(0, 16, zrow, 0)

    def zcp(i, _):
        pltpu.sync_copy(zero_v, acc_sp.at[pl.ds(s * TILE_OUT + i * 16, 16)])
        return 0

    lax.fori_loop(0, TILE_OUT // 16, zcp, 0)
    plsc.subcore_barrier()

    def chunk(k, _):
        ch = s + k * NSUB

        @pl.when(ch < NCH)
        def _run():
            off = ch * CM
            pltpu.sync_copy(src2_hbm.at[pl.ds(off, CM)], srcv)

            @pl.when(c == 0)
            def _():
                pltpu.sync_copy(idx02_hbm.at[pl.ds(off, CM)], idxv)

            @pl.when(c == 1)
            def _():
                pltpu.sync_copy(idx12_hbm.at[pl.ds(off, CM)], idxv)

            rows = (rows0, rows1)
            hb = (h0, h1)
            gath = {
                0: (pltpu.async_copy(nn_hbm.at[srcv.at[pl.ds(0, GW)]],
                                     rows0, gsem),
                    pltpu.async_copy(h_hbm.at[pl.ds(off // 2, GW // 2)],
                                     h0, gsem))
            }
            scat = {}
            for j in range(GJ):
                p = j & 1
                if j < GJ - 1:
                    if j >= 1:
                        scat.pop(j - 1).wait()
                    gath[j + 1] = (
                        pltpu.async_copy(
                            nn_hbm.at[srcv.at[pl.ds((j + 1) * GW, GW)]],
                            rows[1 - p], gsem),
                        pltpu.async_copy(
                            h_hbm.at[pl.ds(off // 2 + (j + 1) * (GW // 2),
                                           GW // 2)],
                            hb[1 - p], gsem))
                ga, gb = gath.pop(j)
                ga.wait()
                gb.wait()
                rp, hp = rows[p], hb[p]

                @plsc.parallel_loop(0, GW // 2, unroll=4)
                def _mul(i2):
                    for jj2 in range(2 * D // 16):
                        r = i2 * 2 + (jj2 // 4)
                        sl = pl.ds((jj2 % 4) * 16, 16)
                        rp[r, sl] = rp[r, sl] * hp[i2, pl.ds(jj2 * 16, 16)]
                scat[j] = pltpu.async_copy(
                    rp, acc_sp.at[idxv.at[pl.ds(j * GW, GW)]],
                    ssem, add=True)
            scat.pop(GJ - 2).wait()
            scat.pop(GJ - 1).wait()
            assert not gath and not scat

        return 0

    lax.fori_loop(0, KMAX, chunk, 0)
    plsc.subcore_barrier()
    nhi = HALF - 8 * 1563

    @pl.when(s < 8)
    def _lo():
        base = s * 1563
        pltpu.sync_copy(acc_sp.at[pl.ds(base, 1563)],
                        outp_hbm.at[pl.ds(c * HALF + base, 1563)])

    @pl.when(s >= 8)
    def _hi():
        base = 8 * 1563 + (s - 8) * (nhi // 8)
        pltpu.sync_copy(acc_sp.at[pl.ds(base, nhi // 8)],
                        outp_hbm.at[pl.ds(c * HALF + base, nhi // 8)])


def kernel(rbf, new_node, W1, b1, W2, b2, src, dst, cluster_id):
    h = _mlp(rbf, W1, b1, W2, b2)
    idx0, idx1 = _mask_kernel(src, dst, cluster_id)
    return _main_kernel(h, new_node, src, idx0, idx1)
